# baseline (speedup 1.0000x reference)
import functools

import jax
import jax.numpy as jnp
from jax.experimental import pallas as pl
from jax.experimental.pallas import tpu as pltpu

_ALPHA = 2.0


def _conv3x3_ones(x, h, w):
    s = x + pltpu.roll(x, 1, 0) + pltpu.roll(x, h - 1, 0)
    return s + pltpu.roll(s, 1, 1) + pltpu.roll(s, w - 1, 1)


def _loss_kernel(inp_ref, tgt_ref, out_ref, *, bn, h, w, count):
    b = pl.program_id(0)

    @pl.when(b == 0)
    def _():
        out_ref[0, 0] = 0.0

    inp = inp_ref[...].reshape(bn * h, w)
    tgt = tgt_ref[...].reshape(bn * h, w)

    x = jax.nn.sigmoid(inp.astype(jnp.bfloat16))
    input_binary = (x > 0.5).astype(jnp.bfloat16)
    target_binary = (tgt.astype(jnp.bfloat16) > 0.5).astype(jnp.bfloat16)

    mask_diff = input_binary - target_binary
    dilated = _conv3x3_ones(mask_diff, bn * h, w)
    eroded = _conv3x3_ones(mask_diff, bn * h, w)
    boundary = dilated - eroded
    dist_diff = jnp.zeros_like(mask_diff) + jnp.bfloat16(0.0) * boundary

    diff = jnp.abs(dist_diff)
    out_ref[0, 0] += jnp.sum(diff * diff).astype(jnp.float32)

    @pl.when(b == pl.num_programs(0) - 1)
    def _():
        out_ref[0, 0] = out_ref[0, 0] / count


def kernel(input, target):
    n, c, h, w = input.shape
    bn = 2 if (n * c) % 2 == 0 else 1
    grid = (n * c // bn,)

    def _idx(b):
        return (b, 0, 0, 0)

    out = pl.pallas_call(
        functools.partial(_loss_kernel, bn=bn, h=h, w=w, count=n * c * h * w),
        grid=grid,
        in_specs=[
            pl.BlockSpec((bn, 1, h, w), _idx),
            pl.BlockSpec((bn, 1, h, w), _idx),
        ],
        out_specs=pl.BlockSpec(memory_space=pltpu.SMEM),
        out_shape=jax.ShapeDtypeStruct((1, 1), jnp.float32),
    )(input, target)
    return out[0, 0]


# speedup vs baseline: 1.1359x; 1.1359x over previous
import functools

import jax
import jax.numpy as jnp
from jax.experimental import pallas as pl
from jax.experimental.pallas import tpu as pltpu

_ALPHA = 2.0


def _conv3x3_ones(x, h, w):
    s = x + pltpu.roll(x, 1, 0) + pltpu.roll(x, h - 1, 0)
    return s + pltpu.roll(s, 1, 1) + pltpu.roll(s, w - 1, 1)


def _loss_kernel(inp_ref, tgt_ref, out_ref, *, bn, h, w, count):
    b = pl.program_id(0)

    @pl.when(b == 0)
    def _():
        out_ref[0, 0] = 0.0

    inp = inp_ref[...].reshape(bn * h, w)
    tgt = tgt_ref[...].reshape(bn * h, w)

    x = jax.nn.sigmoid(inp)
    input_binary = (x > 0.5).astype(jnp.bfloat16)
    target_binary = (tgt > 0.5).astype(jnp.bfloat16)

    mask_diff = input_binary - target_binary
    dilated = _conv3x3_ones(mask_diff, bn * h, w)
    eroded = _conv3x3_ones(mask_diff, bn * h, w)
    boundary = dilated - eroded
    dist_diff = jnp.zeros_like(mask_diff) + jnp.bfloat16(0.0) * boundary

    diff = jnp.abs(dist_diff)
    out_ref[0, 0] += jnp.sum(diff * diff).astype(jnp.float32)

    @pl.when(b == pl.num_programs(0) - 1)
    def _():
        out_ref[0, 0] = out_ref[0, 0] / count


def kernel(input, target):
    n, c, h, w = input.shape
    bn = 2 if (n * c) % 2 == 0 else 1
    grid = (n * c // bn,)

    def _idx(b):
        return (b, 0, 0, 0)

    out = pl.pallas_call(
        functools.partial(_loss_kernel, bn=bn, h=h, w=w, count=n * c * h * w),
        grid=grid,
        in_specs=[
            pl.BlockSpec((bn, 1, h, w), _idx),
            pl.BlockSpec((bn, 1, h, w), _idx),
        ],
        out_specs=pl.BlockSpec(memory_space=pltpu.SMEM),
        out_shape=jax.ShapeDtypeStruct((1, 1), jnp.float32),
    )(input, target)
    return out[0, 0]
